# baseline (speedup 1.0000x reference)
import jax
import jax.numpy as jnp
from jax import lax
from jax.experimental import pallas as pl
from jax.experimental.pallas import tpu as pltpu
from jax.experimental.pallas import tpu_sc as plsc

N_NODE = 10000
N_HE = 10000
D = 128
R = 10112
NC = 2
NS = 16
NW = NC * NS
CHUNK = 128
NNZ = 320000
K = 80
NNZ_PAD = NW * K * CHUNK
RPS = R // NS
_SLICES = ((0, 128), (128, 128), (256, 128), (384, 128), (512, 120))

_F32 = jnp.float32
_I32 = jnp.int32
_MESH = plsc.VectorSubcoreMesh(core_axis_name="c", subcore_axis_name="s")


def _zero_rows(ref, nrows, width):
    @pl.loop(0, nrows)
    def _(i):
        @pl.loop(0, width, step=16)
        def _(k):
            ref.at[i, pl.ds(k, 16)][...] = jnp.zeros((16,), _F32)


def _histk_body(idx_hbm, dep_hbm, out_hbm, idxv, buf, accum):
    c = lax.axis_index("c")
    s = lax.axis_index("s")
    wid = s * NC + c

    _zero_rows(buf, CHUNK, D)
    for off, n in _SLICES:
        pltpu.sync_copy(buf.at[pl.ds(0, n)],
                        accum.at[pl.ds(s * RPS + off, n)])
    pltpu.sync_copy(idx_hbm.at[wid], idxv)

    @pl.loop(0, CHUNK)
    def _(i):
        @pl.loop(0, D, step=16)
        def _(kk):
            buf.at[i, pl.ds(kk, 16)][...] = jnp.ones((16,), _F32)

    plsc.subcore_barrier()

    @pl.loop(0, K)
    def _(j):
        pltpu.sync_copy(buf, accum.at[idxv.at[j]], add=True)

    plsc.subcore_barrier()
    for off, n in _SLICES:
        pltpu.sync_copy(accum.at[pl.ds(s * RPS + off, n)],
                        out_hbm.at[c, pl.ds(s * RPS + off, n)])


_histk = pl.kernel(
    _histk_body,
    out_type=jax.ShapeDtypeStruct((NC, R, D), _F32),
    mesh=_MESH,
    scratch_types=[
        pltpu.VMEM((K, CHUNK), _I32),
        pltpu.VMEM((CHUNK, D), _F32),
        pltpu.VMEM_SHARED((R, D), _F32),
    ],
)


CH = 64
KP = NNZ_PAD // (NW * CH)
DEPTH = 5


def _pass_body(table_hbm, src_hbm, dst_hbm, dep_hbm, out_hbm, istage, dstage,
               rows, accum, *sems):
    isem = sems[0:DEPTH]
    dsem = sems[DEPTH:2 * DEPTH]
    gsem = sems[2 * DEPTH:3 * DEPTH]
    c = lax.axis_index("c")
    s = lax.axis_index("s")
    wid = s * NC + c

    @pl.loop(0, CH)
    def _(i):
        @pl.loop(0, D, step=16)
        def _(kk):
            rows.at[0, i, pl.ds(kk, 16)][...] = jnp.zeros((16,), _F32)

    for off, n in _SLICES:
        o = off
        left = n
        while left > 0:
            m = min(left, CH)
            pltpu.sync_copy(rows.at[0, pl.ds(0, m)],
                            accum.at[pl.ds(s * RPS + o, m)])
            o += m
            left -= m
    plsc.subcore_barrier()

    for t in range(DEPTH):
        pltpu.async_copy(src_hbm.at[wid, t], istage.at[t], isem[t])
        pltpu.async_copy(dst_hbm.at[wid, t], dstage.at[t], dsem[t])
    for t in range(DEPTH - 1):
        pltpu.make_async_copy(src_hbm.at[wid, t], istage.at[t],
                              isem[t]).wait()
        pltpu.async_copy(table_hbm.at[istage.at[t]], rows.at[t], gsem[t])

    @pl.loop(0, KP // DEPTH)
    def _(jj):
        j0 = DEPTH * jj
        for b in range(DEPTH):
            j = j0 + b
            pltpu.make_async_copy(table_hbm.at[istage.at[b]], rows.at[b],
                                  gsem[b]).wait()
            pltpu.make_async_copy(dst_hbm.at[wid, j], dstage.at[b],
                                  dsem[b]).wait()
            pltpu.sync_copy(rows.at[b], accum.at[dstage.at[b]], add=True)

            @pl.when(j + DEPTH < KP)
            def _():
                pltpu.async_copy(src_hbm.at[wid, j + DEPTH], istage.at[b],
                                 isem[b])
                pltpu.async_copy(dst_hbm.at[wid, j + DEPTH], dstage.at[b],
                                 dsem[b])

            bn = (b + DEPTH - 1) % DEPTH

            @pl.when(j + DEPTH - 1 < KP)
            def _():
                pltpu.make_async_copy(src_hbm.at[wid, j + DEPTH - 1],
                                      istage.at[bn], isem[bn]).wait()
                pltpu.async_copy(table_hbm.at[istage.at[bn]], rows.at[bn],
                                 gsem[bn])

    plsc.subcore_barrier()
    for off, n in _SLICES:
        pltpu.sync_copy(accum.at[pl.ds(s * RPS + off, n)],
                        out_hbm.at[c, pl.ds(s * RPS + off, n)])


_pass = pl.kernel(
    _pass_body,
    out_type=jax.ShapeDtypeStruct((NC, R, D), _F32),
    mesh=_MESH,
    scratch_types=[
        pltpu.VMEM((DEPTH, CH), _I32),
        pltpu.VMEM((DEPTH, CH), _I32),
        pltpu.VMEM((DEPTH, CH, D), _F32),
        pltpu.VMEM_SHARED((R, D), _F32),
    ] + [pltpu.SemaphoreType.DMA] * (3 * DEPTH),
)


def _mm_body(x_ref, w_ref, o_ref):
    o_ref[...] = jnp.dot(x_ref[...], w_ref[...],
                         preferred_element_type=_F32)


_mm = pl.pallas_call(_mm_body, out_shape=jax.ShapeDtypeStruct((R, D), _F32))


def _scale_he_body(p_ref, hist_ref, o_ref):
    b = hist_ref[0, :, 0:1] + hist_ref[1, :, 0:1]
    binv = jnp.where(b > 0, 1.0 / b, 0.0)
    o_ref[...] = binv * (p_ref[0] + p_ref[1])


_scale_he = pl.pallas_call(
    _scale_he_body, out_shape=jax.ShapeDtypeStruct((R, D), _F32))


def _fuse_body(q_ref, hist_ref, b_ref, w_ref, o_ref):
    d = hist_ref[0, :, 0:1] + hist_ref[1, :, 0:1]
    dinv = jnp.where(d > 0, 1.0 / d, 0.0)
    h = dinv * (q_ref[0] + q_ref[1]) + b_ref[...]
    h = jnp.where(h > 0, h, jnp.exp(h) - 1.0)
    o_ref[...] = jnp.dot(h, w_ref[...], preferred_element_type=_F32)


_fuse = pl.pallas_call(
    _fuse_body, out_shape=jax.ShapeDtypeStruct((R, D), _F32))


def _final_body(q_ref, hist_ref, b_ref, o_ref):
    d = hist_ref[0, :, 0:1] + hist_ref[1, :, 0:1]
    dinv = jnp.where(d > 0, 1.0 / d, 0.0)
    h = dinv * (q_ref[0] + q_ref[1]) + b_ref[...]
    o_ref[...] = jnp.where(h > 0, h, jnp.exp(h) - 1.0)


_final = pl.pallas_call(
    _final_body, out_shape=jax.ShapeDtypeStruct((R, D), _F32))


def kernel(x, edge_index, W1, b1, W2, b2):
    node = edge_index[0].astype(_I32)
    he = edge_index[1].astype(_I32)
    npad = NNZ_PAD - NNZ
    node_p = jnp.concatenate([node, jnp.full((npad,), N_NODE, _I32)])
    he_p = jnp.concatenate([he, jnp.full((npad,), N_HE, _I32)])
    node_w = node_p.reshape(NW, K, CHUNK)
    he_w = he_p.reshape(NW, K, CHUNK)
    node_w4 = node_p.reshape(NW, KP, CH)
    he_w4 = he_p.reshape(NW, KP, CH)
    xp = jnp.pad(x, ((0, R - N_NODE), (0, 0)))
    b1r = b1.reshape(1, D)
    b2r = b2.reshape(1, D)

    hist_node = _histk(node_w, W1)
    hist_he = _histk(he_w, hist_node)

    xw1 = _mm(xp, W1)
    p1 = _pass(xw1, node_w4, he_w4, hist_he)
    he1 = _scale_he(p1, hist_he)
    q1 = _pass(he1, he_w4, node_w4, p1)
    xw2 = _fuse(q1, hist_node, b1r, W2)
    p2 = _pass(xw2, node_w4, he_w4, q1)
    he2 = _scale_he(p2, hist_he)
    q2 = _pass(he2, he_w4, node_w4, p2)
    out = _final(q2, hist_node, b2r)
    return out[:N_NODE]


# speedup vs baseline: 1.0266x; 1.0266x over previous
import jax
import jax.numpy as jnp
from jax import lax
from jax.experimental import pallas as pl
from jax.experimental.pallas import tpu as pltpu
from jax.experimental.pallas import tpu_sc as plsc

N_NODE = 10000
N_HE = 10000
D = 128
R = 10112
NC = 2
NS = 16
NW = NC * NS
CHUNK = 128
NNZ = 320000
K = 80
NNZ_PAD = NW * K * CHUNK
RPS = R // NS
_SLICES = ((0, 128), (128, 128), (256, 128), (384, 128), (512, 120))

_F32 = jnp.float32
_I32 = jnp.int32
_MESH = plsc.VectorSubcoreMesh(core_axis_name="c", subcore_axis_name="s")


def _zero_rows(ref, nrows, width):
    @pl.loop(0, nrows)
    def _(i):
        @pl.loop(0, width, step=16)
        def _(k):
            ref.at[i, pl.ds(k, 16)][...] = jnp.zeros((16,), _F32)


def _histk_body(idx_hbm, dep_hbm, out_hbm, idxv, buf, accum):
    c = lax.axis_index("c")
    s = lax.axis_index("s")
    wid = s * NC + c

    _zero_rows(buf, CHUNK, D)
    for off, n in _SLICES:
        pltpu.sync_copy(buf.at[pl.ds(0, n)],
                        accum.at[pl.ds(s * RPS + off, n)])
    pltpu.sync_copy(idx_hbm.at[wid], idxv)

    @pl.loop(0, CHUNK)
    def _(i):
        @pl.loop(0, D, step=16)
        def _(kk):
            buf.at[i, pl.ds(kk, 16)][...] = jnp.ones((16,), _F32)

    plsc.subcore_barrier()

    @pl.loop(0, K)
    def _(j):
        pltpu.sync_copy(buf, accum.at[idxv.at[j]], add=True)

    plsc.subcore_barrier()
    for off, n in _SLICES:
        pltpu.sync_copy(accum.at[pl.ds(s * RPS + off, n)],
                        out_hbm.at[c, pl.ds(s * RPS + off, n)])


_histk = pl.kernel(
    _histk_body,
    out_type=jax.ShapeDtypeStruct((NC, R, D), _F32),
    mesh=_MESH,
    scratch_types=[
        pltpu.VMEM((K, CHUNK), _I32),
        pltpu.VMEM((CHUNK, D), _F32),
        pltpu.VMEM_SHARED((R, D), _F32),
    ],
)


CH = 128
KP = NNZ_PAD // (NW * CH)
DEPTH = 2


def _pass_body(table_hbm, src_hbm, dst_hbm, dep_hbm, out_hbm, istage, dstage,
               rows, accum, *sems):
    isem = sems[0:DEPTH]
    dsem = sems[DEPTH:2 * DEPTH]
    gsem = sems[2 * DEPTH:3 * DEPTH]
    c = lax.axis_index("c")
    s = lax.axis_index("s")
    wid = s * NC + c

    @pl.loop(0, CH)
    def _(i):
        @pl.loop(0, D, step=16)
        def _(kk):
            rows.at[0, i, pl.ds(kk, 16)][...] = jnp.zeros((16,), _F32)

    for off, n in _SLICES:
        o = off
        left = n
        while left > 0:
            m = min(left, CH)
            pltpu.sync_copy(rows.at[0, pl.ds(0, m)],
                            accum.at[pl.ds(s * RPS + o, m)])
            o += m
            left -= m
    plsc.subcore_barrier()

    for t in range(DEPTH):
        pltpu.async_copy(src_hbm.at[wid, t], istage.at[t], isem[t])
        pltpu.async_copy(dst_hbm.at[wid, t], dstage.at[t], dsem[t])
    for t in range(DEPTH - 1):
        pltpu.make_async_copy(src_hbm.at[wid, t], istage.at[t],
                              isem[t]).wait()
        pltpu.async_copy(table_hbm.at[istage.at[t]], rows.at[t], gsem[t])

    @pl.loop(0, KP // DEPTH)
    def _(jj):
        j0 = DEPTH * jj
        for b in range(DEPTH):
            j = j0 + b
            pltpu.make_async_copy(table_hbm.at[istage.at[b]], rows.at[b],
                                  gsem[b]).wait()
            pltpu.make_async_copy(dst_hbm.at[wid, j], dstage.at[b],
                                  dsem[b]).wait()
            pltpu.sync_copy(rows.at[b], accum.at[dstage.at[b]], add=True)

            @pl.when(j + DEPTH < KP)
            def _():
                pltpu.async_copy(src_hbm.at[wid, j + DEPTH], istage.at[b],
                                 isem[b])
                pltpu.async_copy(dst_hbm.at[wid, j + DEPTH], dstage.at[b],
                                 dsem[b])

            bn = (b + DEPTH - 1) % DEPTH

            @pl.when(j + DEPTH - 1 < KP)
            def _():
                pltpu.make_async_copy(src_hbm.at[wid, j + DEPTH - 1],
                                      istage.at[bn], isem[bn]).wait()
                pltpu.async_copy(table_hbm.at[istage.at[bn]], rows.at[bn],
                                 gsem[bn])

    plsc.subcore_barrier()
    for off, n in _SLICES:
        pltpu.sync_copy(accum.at[pl.ds(s * RPS + off, n)],
                        out_hbm.at[c, pl.ds(s * RPS + off, n)])


_pass = pl.kernel(
    _pass_body,
    out_type=jax.ShapeDtypeStruct((NC, R, D), _F32),
    mesh=_MESH,
    scratch_types=[
        pltpu.VMEM((DEPTH, CH), _I32),
        pltpu.VMEM((DEPTH, CH), _I32),
        pltpu.VMEM((DEPTH, CH, D), _F32),
        pltpu.VMEM_SHARED((R, D), _F32),
    ] + [pltpu.SemaphoreType.DMA] * (3 * DEPTH),
)


def _mm_body(x_ref, w_ref, o_ref):
    o_ref[...] = jnp.dot(x_ref[...], w_ref[...],
                         preferred_element_type=_F32)


_mm = pl.pallas_call(_mm_body, out_shape=jax.ShapeDtypeStruct((R, D), _F32))


def _scale_he_body(p_ref, hist_ref, o_ref):
    b = hist_ref[0, :, 0:1] + hist_ref[1, :, 0:1]
    binv = jnp.where(b > 0, 1.0 / b, 0.0)
    o_ref[...] = binv * (p_ref[0] + p_ref[1])


_scale_he = pl.pallas_call(
    _scale_he_body, out_shape=jax.ShapeDtypeStruct((R, D), _F32))


def _fuse_body(q_ref, hist_ref, b_ref, w_ref, o_ref):
    d = hist_ref[0, :, 0:1] + hist_ref[1, :, 0:1]
    dinv = jnp.where(d > 0, 1.0 / d, 0.0)
    h = dinv * (q_ref[0] + q_ref[1]) + b_ref[...]
    h = jnp.where(h > 0, h, jnp.exp(h) - 1.0)
    o_ref[...] = jnp.dot(h, w_ref[...], preferred_element_type=_F32)


_fuse = pl.pallas_call(
    _fuse_body, out_shape=jax.ShapeDtypeStruct((R, D), _F32))


def _final_body(q_ref, hist_ref, b_ref, o_ref):
    d = hist_ref[0, :, 0:1] + hist_ref[1, :, 0:1]
    dinv = jnp.where(d > 0, 1.0 / d, 0.0)
    h = dinv * (q_ref[0] + q_ref[1]) + b_ref[...]
    o_ref[...] = jnp.where(h > 0, h, jnp.exp(h) - 1.0)


_final = pl.pallas_call(
    _final_body, out_shape=jax.ShapeDtypeStruct((R, D), _F32))


def kernel(x, edge_index, W1, b1, W2, b2):
    node = edge_index[0].astype(_I32)
    he = edge_index[1].astype(_I32)
    npad = NNZ_PAD - NNZ
    node_p = jnp.concatenate([node, jnp.full((npad,), N_NODE, _I32)])
    he_p = jnp.concatenate([he, jnp.full((npad,), N_HE, _I32)])
    node_w = node_p.reshape(NW, K, CHUNK)
    he_w = he_p.reshape(NW, K, CHUNK)
    node_w4 = node_p.reshape(NW, KP, CH)
    he_w4 = he_p.reshape(NW, KP, CH)
    xp = jnp.pad(x, ((0, R - N_NODE), (0, 0)))
    b1r = b1.reshape(1, D)
    b2r = b2.reshape(1, D)

    hist_node = _histk(node_w, W1)
    hist_he = _histk(he_w, hist_node)

    xw1 = _mm(xp, W1)
    p1 = _pass(xw1, node_w4, he_w4, hist_he)
    he1 = _scale_he(p1, hist_he)
    q1 = _pass(he1, he_w4, node_w4, p1)
    xw2 = _fuse(q1, hist_node, b1r, W2)
    p2 = _pass(xw2, node_w4, he_w4, q1)
    he2 = _scale_he(p2, hist_he)
    q2 = _pass(he2, he_w4, node_w4, p2)
    out = _final(q2, hist_node, b2r)
    return out[:N_NODE]


# speedup vs baseline: 1.1683x; 1.1380x over previous
import jax
import jax.numpy as jnp
from jax import lax
from jax.experimental import pallas as pl
from jax.experimental.pallas import tpu as pltpu
from jax.experimental.pallas import tpu_sc as plsc

N_NODE = 10000
N_HE = 10000
D = 128
R = 10112
NC = 2
NS = 16
NW = NC * NS
CHUNK = 128
NNZ = 320000
K = 80
NNZ_PAD = NW * K * CHUNK
RPS = R // NS
_SLICES = ((0, 128), (128, 128), (256, 128), (384, 128), (512, 120))

_F32 = jnp.float32
_I32 = jnp.int32
_MESH = plsc.VectorSubcoreMesh(core_axis_name="c", subcore_axis_name="s")


def _zero_rows(ref, nrows, width):
    @pl.loop(0, nrows)
    def _(i):
        @pl.loop(0, width, step=16)
        def _(k):
            ref.at[i, pl.ds(k, 16)][...] = jnp.zeros((16,), _F32)


def _histk_body(idx_hbm, dep_hbm, out_hbm, idxv, buf, accum):
    c = lax.axis_index("c")
    s = lax.axis_index("s")
    wid = s * NC + c

    _zero_rows(buf, CHUNK, D)
    for off, n in _SLICES:
        pltpu.sync_copy(buf.at[pl.ds(0, n)],
                        accum.at[pl.ds(s * RPS + off, n)])
    pltpu.sync_copy(idx_hbm.at[wid], idxv)

    @pl.loop(0, CHUNK)
    def _(i):
        @pl.loop(0, D, step=16)
        def _(kk):
            buf.at[i, pl.ds(kk, 16)][...] = jnp.ones((16,), _F32)

    plsc.subcore_barrier()

    @pl.loop(0, K)
    def _(j):
        pltpu.sync_copy(buf, accum.at[idxv.at[j]], add=True)

    plsc.subcore_barrier()
    for off, n in _SLICES:
        pltpu.sync_copy(accum.at[pl.ds(s * RPS + off, n)],
                        out_hbm.at[c, pl.ds(s * RPS + off, n)])


_histk = pl.kernel(
    _histk_body,
    out_type=jax.ShapeDtypeStruct((NC, R, D), _F32),
    mesh=_MESH,
    scratch_types=[
        pltpu.VMEM((K, CHUNK), _I32),
        pltpu.VMEM((CHUNK, D), _F32),
        pltpu.VMEM_SHARED((R, D), _F32),
    ],
)


CH = 128
KP = NNZ_PAD // (NW * CH)
DEPTH = 3


def _pass_body(table_hbm, src_hbm, dst_hbm, dep_hbm, out_hbm, istage, dstage,
               rows, accum, *sems):
    isem = sems[0:DEPTH]
    dsem = sems[DEPTH:2 * DEPTH]
    gsem = sems[2 * DEPTH:3 * DEPTH]
    c = lax.axis_index("c")
    s = lax.axis_index("s")
    wid = s * NC + c

    @pl.loop(0, CH)
    def _(i):
        @pl.loop(0, D, step=16)
        def _(kk):
            rows.at[0, i, pl.ds(kk, 16)][...] = jnp.zeros((16,), _F32)

    for off, n in _SLICES:
        o = off
        left = n
        while left > 0:
            m = min(left, CH)
            pltpu.sync_copy(rows.at[0, pl.ds(0, m)],
                            accum.at[pl.ds(s * RPS + o, m)])
            o += m
            left -= m
    plsc.subcore_barrier()

    for t in range(DEPTH):
        pltpu.async_copy(src_hbm.at[wid, t], istage.at[t], isem[t])
        pltpu.async_copy(dst_hbm.at[wid, t], dstage.at[t], dsem[t])
    for t in range(DEPTH - 1):
        pltpu.make_async_copy(src_hbm.at[wid, t], istage.at[t],
                              isem[t]).wait()
        pltpu.async_copy(table_hbm.at[istage.at[t]], rows.at[t], gsem[t])

    @pl.loop(0, (KP + DEPTH - 1) // DEPTH)
    def _(jj):
        j0 = DEPTH * jj
        for b in range(DEPTH):
            j = j0 + b

            @pl.when(j < KP)
            def _():
                pltpu.make_async_copy(table_hbm.at[istage.at[b]], rows.at[b],
                                      gsem[b]).wait()
                pltpu.make_async_copy(dst_hbm.at[wid, j], dstage.at[b],
                                      dsem[b]).wait()
                pltpu.sync_copy(rows.at[b], accum.at[dstage.at[b]], add=True)

                @pl.when(j + DEPTH < KP)
                def _():
                    pltpu.async_copy(src_hbm.at[wid, j + DEPTH],
                                     istage.at[b], isem[b])
                    pltpu.async_copy(dst_hbm.at[wid, j + DEPTH],
                                     dstage.at[b], dsem[b])

                bn = (b + DEPTH - 1) % DEPTH

                @pl.when(j + DEPTH - 1 < KP)
                def _():
                    pltpu.make_async_copy(src_hbm.at[wid, j + DEPTH - 1],
                                          istage.at[bn], isem[bn]).wait()
                    pltpu.async_copy(table_hbm.at[istage.at[bn]],
                                     rows.at[bn], gsem[bn])

    plsc.subcore_barrier()
    for off, n in _SLICES:
        pltpu.sync_copy(accum.at[pl.ds(s * RPS + off, n)],
                        out_hbm.at[c, pl.ds(s * RPS + off, n)])


_pass = pl.kernel(
    _pass_body,
    out_type=jax.ShapeDtypeStruct((NC, R, D), _F32),
    mesh=_MESH,
    scratch_types=[
        pltpu.VMEM((DEPTH, CH), _I32),
        pltpu.VMEM((DEPTH, CH), _I32),
        pltpu.VMEM((DEPTH, CH, D), _F32),
        pltpu.VMEM_SHARED((R, D), _F32),
    ] + [pltpu.SemaphoreType.DMA] * (3 * DEPTH),
)


def _mm_body(x_ref, w_ref, o_ref):
    o_ref[...] = jnp.dot(x_ref[...], w_ref[...],
                         preferred_element_type=_F32)


_mm = pl.pallas_call(_mm_body, out_shape=jax.ShapeDtypeStruct((R, D), _F32))


def _scale_he_body(p_ref, hist_ref, o_ref):
    b = hist_ref[0, :, 0:1] + hist_ref[1, :, 0:1]
    binv = jnp.where(b > 0, 1.0 / b, 0.0)
    o_ref[...] = binv * (p_ref[0] + p_ref[1])


_scale_he = pl.pallas_call(
    _scale_he_body, out_shape=jax.ShapeDtypeStruct((R, D), _F32))


def _fuse_body(q_ref, hist_ref, b_ref, w_ref, o_ref):
    d = hist_ref[0, :, 0:1] + hist_ref[1, :, 0:1]
    dinv = jnp.where(d > 0, 1.0 / d, 0.0)
    h = dinv * (q_ref[0] + q_ref[1]) + b_ref[...]
    h = jnp.where(h > 0, h, jnp.exp(h) - 1.0)
    o_ref[...] = jnp.dot(h, w_ref[...], preferred_element_type=_F32)


_fuse = pl.pallas_call(
    _fuse_body, out_shape=jax.ShapeDtypeStruct((R, D), _F32))


def _final_body(q_ref, hist_ref, b_ref, o_ref):
    d = hist_ref[0, :, 0:1] + hist_ref[1, :, 0:1]
    dinv = jnp.where(d > 0, 1.0 / d, 0.0)
    h = dinv * (q_ref[0] + q_ref[1]) + b_ref[...]
    o_ref[...] = jnp.where(h > 0, h, jnp.exp(h) - 1.0)


_final = pl.pallas_call(
    _final_body, out_shape=jax.ShapeDtypeStruct((R, D), _F32))


def kernel(x, edge_index, W1, b1, W2, b2):
    node = edge_index[0].astype(_I32)
    he = edge_index[1].astype(_I32)
    npad = NNZ_PAD - NNZ
    node_p = jnp.concatenate([node, jnp.full((npad,), N_NODE, _I32)])
    he_p = jnp.concatenate([he, jnp.full((npad,), N_HE, _I32)])
    node_w = node_p.reshape(NW, K, CHUNK)
    he_w = he_p.reshape(NW, K, CHUNK)
    node_w4 = node_p.reshape(NW, KP, CH)
    he_w4 = he_p.reshape(NW, KP, CH)
    xp = jnp.pad(x, ((0, R - N_NODE), (0, 0)))
    b1r = b1.reshape(1, D)
    b2r = b2.reshape(1, D)

    hist_node = _histk(node_w, W1)
    hist_he = _histk(he_w, hist_node)

    xw1 = _mm(xp, W1)
    p1 = _pass(xw1, node_w4, he_w4, hist_he)
    he1 = _scale_he(p1, hist_he)
    q1 = _pass(he1, he_w4, node_w4, p1)
    xw2 = _fuse(q1, hist_node, b1r, W2)
    p2 = _pass(xw2, node_w4, he_w4, q1)
    he2 = _scale_he(p2, hist_he)
    q2 = _pass(he2, he_w4, node_w4, p2)
    out = _final(q2, hist_node, b2r)
    return out[:N_NODE]
